# baseline (speedup 1.0000x reference)
import functools

import jax
import jax.numpy as jnp
from jax import lax
from jax.experimental import pallas as pl
from jax.experimental.pallas import tpu as pltpu
from jax.experimental.pallas import tpu_sc as plsc

N = 32768
D = 128
B = 16
CHUNK = 256
UNROLL = 8
NVEC = D // 16


def _segment_max_body(values_hbm, ps_hbm, out_hbm,
                      ps_v, buf, ebuf, acc_v, stage_v, merge_v,
                      outbuf_v, shared, sem0, sem1):
    c = lax.axis_index("c")
    s = lax.axis_index("s")

    ps_v[pl.ds(16, 16)] = jnp.full((16,), N, dtype=jnp.int32)
    pltpu.sync_copy(ps_hbm, ps_v.at[pl.ds(0, B + 1)])
    ps_lo16 = ps_v[pl.ds(0, 16)]
    iota = lax.iota(jnp.int32, 16)

    core_vec = ps_v[pl.ds(c * 8, 16)]
    lo_c = core_vec[0]
    hi_c = core_vec[8]
    len_c = hi_c - lo_c
    lo_w = lo_c + ((s * len_c) >> 4)
    hi_w = lo_c + (((s + 1) * len_c) >> 4)

    s0 = jnp.int32(0)
    for k in range(1, 16):
        s0 = s0 + (ps_lo16[k] <= lo_w).astype(jnp.int32)
    bvec = ps_v[pl.ds(s0 + 1, 16)]
    p1 = jnp.clip(bvec[0], lo_w, hi_w)
    p2 = jnp.clip(bvec[1], lo_w, hi_w)

    lo_wa = (lo_w + 7) & -8
    hi_wa = hi_w & -8
    n_blocks = (hi_wa - lo_wa + (CHUNK - 1)) >> 8

    neg_inf = jnp.full((16,), -jnp.inf, dtype=jnp.float32)
    for t in range(3):
        for j in range(NVEC):
            acc_v[t, pl.ds(16 * j, 16)] = neg_inf

    sems = (sem0, sem1)

    def blk_base(i):
        off = jnp.minimum(lo_wa + i * CHUNK, N - CHUNK)
        return pl.multiple_of(off, 8)

    def start_dma(i, parity):
        pltpu.make_async_copy(
            values_hbm.at[pl.ds(blk_base(i), CHUNK)],
            buf.at[parity], sems[parity]).start()

    def process(i, parity):
        pltpu.make_async_copy(
            values_hbm.at[pl.ds(0, CHUNK)],
            buf.at[parity], sems[parity]).wait()
        cbuf = buf.at[parity]
        base = blk_base(i)
        lo_row = (lo_wa + i * CHUNK) - base
        blk_hi = jnp.minimum(CHUNK, hi_wa - base)
        q1 = jnp.clip(p1 - base, lo_row, blk_hi)
        q2 = jnp.clip(p2 - base, lo_row, blk_hi)

        def run_sub(t, lo_r, hi_r):
            acc = tuple(acc_v[t, pl.ds(16 * j, 16)] for j in range(NVEC))
            cnt = hi_r - lo_r
            nmain = cnt >> 3

            def mbody(it, a):
                r0 = lo_r + it * UNROLL
                for u in range(UNROLL):
                    a = tuple(
                        jnp.maximum(a[j], cbuf[r0 + u, pl.ds(16 * j, 16)])
                        for j in range(NVEC)
                    )
                return a

            acc = lax.fori_loop(0, nmain, mbody, acc)

            def rbody(r, a):
                return tuple(
                    jnp.maximum(a[j], cbuf[r, pl.ds(16 * j, 16)])
                    for j in range(NVEC)
                )

            acc = lax.fori_loop(lo_r + (nmain << 3), hi_r, rbody, acc)
            for j in range(NVEC):
                acc_v[t, pl.ds(16 * j, 16)] = acc[j]

        run_sub(0, lo_row, q1)
        run_sub(1, q1, q2)
        run_sub(2, q2, blk_hi)

    start_dma(0, 0)
    pl.when(1 < n_blocks)(lambda: start_dma(1, 1))

    def pair_body(k, carry):
        process(2 * k, 0)
        pl.when(2 * k + 2 < n_blocks)(lambda: start_dma(2 * k + 2, 0))
        process(2 * k + 1, 1)
        pl.when(2 * k + 3 < n_blocks)(lambda: start_dma(2 * k + 3, 1))
        return carry

    lax.fori_loop(0, n_blocks >> 1, pair_body, 0)
    last = n_blocks - 1
    pl.when((n_blocks & 1) == 1)(lambda: process(last, 0))

    def edge(off, wlo, whi):
        off = pl.multiple_of(off, 8)
        pltpu.sync_copy(values_hbm.at[pl.ds(off, 8)], ebuf)
        accs = [[acc_v[t, pl.ds(16 * j, 16)] for j in range(NVEC)]
                for t in range(3)]
        for r in range(8):
            g = off + r
            inw = jnp.logical_and(g >= wlo, g < whi)
            ge1 = g >= p1
            ge2 = g >= p2
            preds = (
                jnp.logical_and(inw, jnp.logical_not(ge1)),
                jnp.logical_and(inw, jnp.logical_and(ge1, jnp.logical_not(ge2))),
                jnp.logical_and(inw, ge2),
            )
            for t in range(3):
                for j in range(NVEC):
                    accs[t][j] = jnp.maximum(
                        accs[t][j],
                        jnp.where(preds[t], ebuf[r, pl.ds(16 * j, 16)],
                                  neg_inf))
        for t in range(3):
            for j in range(NVEC):
                acc_v[t, pl.ds(16 * j, 16)] = accs[t][j]

    edge(jnp.maximum(lo_wa - 8, 0), lo_w, lo_wa)
    edge(jnp.minimum(hi_wa, N - 8), hi_wa, hi_w)

    for k in range(8):
        for j in range(NVEC):
            stage_v[k, pl.ds(16 * j, 16)] = neg_inf
    for t in range(3):
        r_t = jnp.clip(s0 + t - c * 8, 0, 7)
        for j in range(NVEC):
            stage_v[r_t, pl.ds(16 * j, 16)] = jnp.maximum(
                stage_v[r_t, pl.ds(16 * j, 16)],
                acc_v[t, pl.ds(16 * j, 16)])
    pltpu.sync_copy(stage_v, shared.at[s])
    plsc.subcore_barrier()

    @pl.when(s == 0)
    def _():
        pltpu.sync_copy(shared, merge_v)
        for k in range(8):
            acc = tuple(merge_v[0, k, pl.ds(16 * j, 16)]
                        for j in range(NVEC))
            for w in range(1, 16):
                acc = tuple(
                    jnp.maximum(acc[j], merge_v[w, k, pl.ds(16 * j, 16)])
                    for j in range(NVEC))
            for j in range(NVEC):
                outbuf_v[k, pl.ds(16 * j, 16)] = acc[j]
        base_seg = pl.multiple_of(c * 8, 8)
        pltpu.sync_copy(outbuf_v, out_hbm.at[pl.ds(base_seg, 8)])


@jax.jit
def kernel(values, prefix_sum):
    run = functools.partial(
        pl.kernel,
        mesh=plsc.VectorSubcoreMesh(core_axis_name="c", subcore_axis_name="s"),
        out_type=jax.ShapeDtypeStruct((B, D), jnp.float32),
        scratch_types=[
            pltpu.VMEM((32,), jnp.int32),
            pltpu.VMEM((2, CHUNK, D), jnp.float32),
            pltpu.VMEM((8, D), jnp.float32),
            pltpu.VMEM((3, D), jnp.float32),
            pltpu.VMEM((8, D), jnp.float32),
            pltpu.VMEM((16, 8, D), jnp.float32),
            pltpu.VMEM((8, D), jnp.float32),
            pltpu.VMEM_SHARED((16, 8, D), jnp.float32),
            pltpu.SemaphoreType.DMA,
            pltpu.SemaphoreType.DMA,
        ],
    )(_segment_max_body)
    return run(values, prefix_sum.astype(jnp.int32))


# speedup vs baseline: 1.3064x; 1.3064x over previous
import functools

import jax
import jax.numpy as jnp
from jax import lax
from jax.experimental import pallas as pl
from jax.experimental.pallas import tpu as pltpu
from jax.experimental.pallas import tpu_sc as plsc

N = 32768
D = 128
B = 16
CHUNK = 256
UNROLL = 8
NVEC = D // 16


def _segment_max_body(values_hbm, ps_hbm, out_hbm,
                      ps_v, buf, ebuf, acc_v, merge_v, outbuf_v, shared,
                      sem0, sem1):
    c = lax.axis_index("c")
    s = lax.axis_index("s")
    b = c * 8 + s // 2
    h = s % 2

    pltpu.sync_copy(ps_hbm, ps_v.at[pl.ds(0, B + 1)])
    ps_vec = ps_v[pl.ds(b, 16)]
    start = ps_vec[0]
    end = ps_vec[1]

    mid = (start + ((end - start) >> 1)) & -8
    lo = jnp.where(h == 0, start, mid)
    hi = jnp.where(h == 0, mid, end)
    lo_a = (lo + 7) & -8
    hi_a = hi & -8
    n_chunks = (hi_a - lo_a + (CHUNK - 1)) >> 8

    neg_inf = jnp.full((16,), -jnp.inf, dtype=jnp.float32)
    for j in range(NVEC):
        acc_v[pl.ds(16 * j, 16)] = neg_inf

    sems = (sem0, sem1)

    def chunk_off(i):
        off = jnp.minimum(lo_a + i * CHUNK, hi_a - CHUNK)
        return pl.multiple_of(off, 8)

    def start_dma(i, parity):
        pltpu.make_async_copy(
            values_hbm.at[pl.ds(chunk_off(i), CHUNK)],
            buf.at[parity], sems[parity]).start()

    def process(i, parity):
        pltpu.make_async_copy(
            values_hbm.at[pl.ds(0, CHUNK)],
            buf.at[parity], sems[parity]).wait()
        cbuf = buf.at[parity]
        acc = tuple(acc_v[pl.ds(16 * j, 16)] for j in range(NVEC))
        lo_row = (lo_a + i * CHUNK) - chunk_off(i)

        def row_body(it, a):
            r0 = it * UNROLL
            for u in range(UNROLL):
                a = tuple(
                    jnp.maximum(a[j], cbuf[r0 + u, pl.ds(16 * j, 16)])
                    for j in range(NVEC)
                )
            return a

        acc = lax.fori_loop(lo_row >> 3, CHUNK // UNROLL, row_body, acc)
        for j in range(NVEC):
            acc_v[pl.ds(16 * j, 16)] = acc[j]

    start_dma(0, 0)
    pl.when(1 < n_chunks)(lambda: start_dma(1, 1))

    def pair_body(k, carry):
        process(2 * k, 0)
        pl.when(2 * k + 2 < n_chunks)(lambda: start_dma(2 * k + 2, 0))
        process(2 * k + 1, 1)
        pl.when(2 * k + 3 < n_chunks)(lambda: start_dma(2 * k + 3, 1))
        return carry

    lax.fori_loop(0, n_chunks >> 1, pair_body, 0)
    last = n_chunks - 1
    pl.when((n_chunks & 1) == 1)(lambda: process(last, 0))

    acc = list(acc_v[pl.ds(16 * j, 16)] for j in range(NVEC))

    def edge(acc, off, row_lo, row_hi):
        off = pl.multiple_of(off, 8)
        pltpu.sync_copy(values_hbm.at[pl.ds(off, 8)], ebuf)
        for r in range(8):
            g = off + r
            pred = jnp.logical_and(g >= row_lo, g < row_hi)
            for j in range(NVEC):
                acc[j] = jnp.maximum(
                    acc[j],
                    jnp.where(pred, ebuf[r, pl.ds(16 * j, 16)], neg_inf))
        return acc

    acc = edge(acc, jnp.maximum(lo_a - 8, 0), lo, lo_a)
    acc = edge(acc, jnp.minimum(hi_a, N - 8), hi_a, hi)

    for j in range(NVEC):
        acc_v[pl.ds(16 * j, 16)] = acc[j]
    pltpu.sync_copy(acc_v, shared.at[s])
    plsc.subcore_barrier()

    @pl.when(s == 0)
    def _():
        pltpu.sync_copy(shared, merge_v)
        for k in range(8):
            for j in range(NVEC):
                outbuf_v[k, pl.ds(16 * j, 16)] = jnp.maximum(
                    merge_v[2 * k, pl.ds(16 * j, 16)],
                    merge_v[2 * k + 1, pl.ds(16 * j, 16)])
        base = pl.multiple_of(c * 8, 8)
        pltpu.sync_copy(outbuf_v, out_hbm.at[pl.ds(base, 8)])


@jax.jit
def kernel(values, prefix_sum):
    run = functools.partial(
        pl.kernel,
        mesh=plsc.VectorSubcoreMesh(core_axis_name="c", subcore_axis_name="s"),
        out_type=jax.ShapeDtypeStruct((B, D), jnp.float32),
        scratch_types=[
            pltpu.VMEM((32,), jnp.int32),
            pltpu.VMEM((2, CHUNK, D), jnp.float32),
            pltpu.VMEM((8, D), jnp.float32),
            pltpu.VMEM((D,), jnp.float32),
            pltpu.VMEM((16, D), jnp.float32),
            pltpu.VMEM((8, D), jnp.float32),
            pltpu.VMEM_SHARED((16, D), jnp.float32),
            pltpu.SemaphoreType.DMA,
            pltpu.SemaphoreType.DMA,
        ],
    )(_segment_max_body)
    return run(values, prefix_sum.astype(jnp.int32))
